# Initial kernel scaffold; baseline (speedup 1.0000x reference)
import jax
import jax.numpy as jnp
from jax.experimental import pallas as pl


def kernel(features, edge_index, W_self1, W_neigh1, b1, W_self2, W_neigh2, b2):
    raise NotImplementedError("write your pallas kernel here")


# speedup vs baseline: 4.0805x; 4.0805x over previous
import functools

import jax
import jax.numpy as jnp
from jax import lax
from jax.experimental import pallas as pl
from jax.experimental.pallas import tpu as pltpu
from jax.experimental.pallas import tpu_sc as plsc

N = 10000
D = 128
NC = 2
NS = 16
NW = NC * NS
K = 128
NPAD = 10112
RPS = NPAD // NS

_mesh = plsc.VectorSubcoreMesh(core_axis_name="c", subcore_axis_name="s")


def _fill_rows(ref, nrows, ncols, value):
    v = jnp.full((16,), value, jnp.float32)

    def row(i, carry):
        for j in range(ncols // 16):
            ref[i, pl.ds(j * 16, 16)] = v
        return carry

    lax.fori_loop(0, nrows, row, 0)


def _zero_shared_slice(buf_v, sh, r0):
    nfull = RPS // K
    rem = RPS - nfull * K
    for j in range(nfull):
        pltpu.sync_copy(buf_v, sh.at[pl.ds(r0 + j * K, K)])
    if rem:
        pltpu.sync_copy(buf_v.at[pl.ds(0, rem)],
                        sh.at[pl.ds(r0 + nfull * K, rem)])


def _copy_out_slice(buf_v, sh, out, c, r0):
    nfull = RPS // K
    rem = RPS - nfull * K
    for j in range(nfull):
        pltpu.sync_copy(sh.at[pl.ds(r0 + j * K, K)], buf_v)
        pltpu.sync_copy(buf_v, out.at[c, pl.ds(r0 + j * K, K)])
    if rem:
        pltpu.sync_copy(sh.at[pl.ds(r0 + nfull * K, rem)],
                        buf_v.at[pl.ds(0, rem)])
        pltpu.sync_copy(buf_v.at[pl.ds(0, rem)],
                        out.at[c, pl.ds(r0 + nfull * K, rem)])


def _make_sc_agg(epad):
    epw = epad // NW
    nchunk = epw // K

    @functools.partial(
        pl.kernel, mesh=_mesh,
        out_type=jax.ShapeDtypeStruct((NC, NPAD, D), jnp.float32),
        scratch_types=(
            pltpu.VMEM((K,), jnp.int32),
            pltpu.VMEM((K,), jnp.int32),
            pltpu.VMEM((K, D), jnp.float32),
            pltpu.VMEM_SHARED((NPAD, D), jnp.float32),
            pltpu.SemaphoreType.DMA,
        ))
    def sc_agg(x_hbm, src_hbm, dst_hbm, agg_out, src_v, dst_v, rows_v,
               agg_sh, sem):
        c = lax.axis_index("c")
        s = lax.axis_index("s")
        wid = c * NS + s
        r0 = s * RPS

        _fill_rows(rows_v, K, D, 0.0)
        _zero_shared_slice(rows_v, agg_sh, r0)
        plsc.subcore_barrier()

        ebase = wid * epw

        def chunk(i, carry):
            e0 = pl.multiple_of(ebase + i * K, K)
            pltpu.sync_copy(src_hbm.at[pl.ds(e0, K)], src_v)
            pltpu.sync_copy(dst_hbm.at[pl.ds(e0, K)], dst_v)
            pltpu.async_copy(x_hbm.at[src_v], rows_v, sem).wait()
            pltpu.sync_copy(rows_v, agg_sh.at[dst_v], add=True)
            return carry

        lax.fori_loop(0, nchunk, chunk, 0)
        plsc.subcore_barrier()
        _copy_out_slice(rows_v, agg_sh, agg_out, c, r0)

    return sc_agg


def _make_sc_deg(epad):
    epw = epad // NW
    nchunk = epw // K

    @functools.partial(
        pl.kernel, mesh=_mesh,
        out_type=jax.ShapeDtypeStruct((NC, NPAD, D), jnp.float32),
        scratch_types=(
            pltpu.VMEM((K,), jnp.int32),
            pltpu.VMEM((K, D), jnp.float32),
            pltpu.VMEM((K, D), jnp.float32),
            pltpu.VMEM_SHARED((NPAD, D), jnp.float32),
        ))
    def sc_deg(dst_hbm, deg_out, dst_v, ones_v, buf_v, deg_sh):
        c = lax.axis_index("c")
        s = lax.axis_index("s")
        wid = c * NS + s
        r0 = s * RPS

        _fill_rows(buf_v, K, D, 0.0)
        _zero_shared_slice(buf_v, deg_sh, r0)
        _fill_rows(ones_v, K, D, 1.0)
        plsc.subcore_barrier()

        ebase = wid * epw

        def chunk(i, carry):
            e0 = pl.multiple_of(ebase + i * K, K)
            pltpu.sync_copy(dst_hbm.at[pl.ds(e0, K)], dst_v)
            pltpu.sync_copy(ones_v, deg_sh.at[dst_v], add=True)
            return carry

        lax.fori_loop(0, nchunk, chunk, 0)
        plsc.subcore_barrier()
        _copy_out_slice(buf_v, deg_sh, deg_out, c, r0)

    return sc_deg


def _tc_layer(x, a0, a1, d0, d1, w_self, w_neigh, b2d, relu):
    bm = 1000

    def body(x_ref, a0_ref, a1_ref, d0_ref, d1_ref, ws_ref, wn_ref, b_ref,
             o_ref):
        deg = jnp.maximum(d0_ref[...] + d1_ref[...], 1.0)
        h = (a0_ref[...] + a1_ref[...]) / deg
        acc = jnp.dot(x_ref[...], ws_ref[...],
                      preferred_element_type=jnp.float32)
        acc = acc + jnp.dot(h, wn_ref[...],
                            preferred_element_type=jnp.float32)
        acc = acc + b_ref[...]
        if relu:
            acc = jnp.maximum(acc, 0.0)
        o_ref[...] = acc

    row_spec = pl.BlockSpec((bm, D), lambda i: (i, 0))
    deg_spec = pl.BlockSpec((bm, 1), lambda i: (i, 0))
    return pl.pallas_call(
        body,
        grid=(N // bm,),
        in_specs=[
            row_spec, row_spec, row_spec, deg_spec, deg_spec,
            pl.BlockSpec((D, D), lambda i: (0, 0)),
            pl.BlockSpec((D, D), lambda i: (0, 0)),
            pl.BlockSpec((1, D), lambda i: (0, 0)),
        ],
        out_specs=row_spec,
        out_shape=jax.ShapeDtypeStruct((N, D), jnp.float32),
    )(x, a0, a1, d0, d1, w_self, w_neigh, b2d)


def kernel(features, edge_index, W_self1, W_neigh1, b1, W_self2, W_neigh2, b2):
    e = edge_index.shape[1]
    epad = -(-e // (NW * K)) * (NW * K)
    src = edge_index[0]
    dst = edge_index[1]
    if epad > e:
        pad = epad - e
        src = jnp.concatenate([src, jnp.zeros((pad,), jnp.int32)])
        dst = jnp.concatenate([dst, jnp.full((pad,), NPAD - 1, jnp.int32)])

    sc_agg = _make_sc_agg(epad)
    sc_deg = _make_sc_deg(epad)

    deg = sc_deg(dst)
    agg1 = sc_agg(features, src, dst)
    d0 = deg[0, :N, 0:1]
    d1 = deg[1, :N, 0:1]
    b1r = b1.reshape(1, D)
    b2r = b2.reshape(1, D)
    h1 = _tc_layer(features, agg1[0, :N], agg1[1, :N], d0, d1,
                   W_self1, W_neigh1, b1r, relu=True)
    agg2 = sc_agg(h1, src, dst)
    out = _tc_layer(h1, agg2[0, :N], agg2[1, :N], d0, d1,
                    W_self2, W_neigh2, b2r, relu=False)
    return out
